# baseline (device time: 17024 ns/iter reference)
import jax
import jax.numpy as jnp
from jax import lax
from jax.experimental import pallas as pl
from jax.experimental.pallas import tpu as pltpu

M = 1024
N = 1024
NH = 512
H = 512
C = 4
CH = H // C


def kernel(x):
    def body(
        x_hbm,
        out_hbm,
        xs_buf,
        send_buf,
        xl_buf,
        lout_buf,
        recv_buf,
        in_sems,
        lin_sem,
        lout_sem,
        xout_sems,
        s1_send,
        s1_recv,
        s2_send,
        s2_recv,
    ):
        p = lax.axis_index("x")
        y = lax.axis_index("y")
        z = lax.axis_index("z")
        q = 1 - p
        r = y % 2
        by = y + 1 - 2 * r

        barrier_sem = pltpu.get_barrier_semaphore()
        pl.semaphore_signal(
            barrier_sem, inc=1, device_id=(q, y, z),
            device_id_type=pl.DeviceIdType.MESH,
        )
        pl.semaphore_signal(
            barrier_sem, inc=1, device_id=(p, by, z),
            device_id_type=pl.DeviceIdType.MESH,
        )

        dma_in = []
        for c in range(C):
            d = pltpu.make_async_copy(
                x_hbm.at[pl.ds(r * H + c * CH, CH), pl.ds(q * NH, NH)],
                xs_buf.at[pl.ds(c * CH, CH)],
                in_sems.at[c],
            )
            d.start()
            dma_in.append(d)
        dma_lin = pltpu.make_async_copy(
            x_hbm.at[:, pl.ds(p * NH, NH)], xl_buf, lin_sem
        )
        dma_lin.start()

        pl.semaphore_wait(barrier_sem, 2)

        rdma1 = []
        for c in range(C):
            dma_in[c].wait()
            send_buf[pl.ds(c * CH, CH)] = xs_buf[pl.ds(c * CH, CH)].astype(
                jnp.bfloat16
            )
            rdma = pltpu.make_async_remote_copy(
                src_ref=send_buf.at[pl.ds(c * CH, CH)],
                dst_ref=recv_buf.at[pl.ds(c * CH, CH)],
                send_sem=s1_send.at[c],
                recv_sem=s1_recv.at[c],
                device_id=(q, y, z),
                device_id_type=pl.DeviceIdType.MESH,
            )
            rdma.start()
            rdma1.append(rdma)

        dma_lin.wait()
        lout_buf[...] = xl_buf[...].astype(jnp.bfloat16)
        dma_lout = pltpu.make_async_copy(
            lout_buf, out_hbm.at[pl.ds(p * M, M)], lout_sem
        )
        dma_lout.start()

        rdma2 = []
        dma_xout = []
        for c in range(C):
            row0 = q * M + r * H + c * CH
            rdma1[c].wait_recv()
            rdma = pltpu.make_async_remote_copy(
                src_ref=recv_buf.at[pl.ds(c * CH, CH)],
                dst_ref=out_hbm.at[pl.ds(row0, CH)],
                send_sem=s2_send.at[c],
                recv_sem=s2_recv.at[c],
                device_id=(p, by, z),
                device_id_type=pl.DeviceIdType.MESH,
            )
            rdma.start()
            rdma2.append(rdma)
            d = pltpu.make_async_copy(
                recv_buf.at[pl.ds(c * CH, CH)],
                out_hbm.at[pl.ds(row0, CH)],
                xout_sems.at[c],
            )
            d.start()
            dma_xout.append(d)

        for c in range(C):
            rdma1[c].wait_send()
            rdma2[c].wait()
            dma_xout[c].wait()
        dma_lout.wait()

    return pl.pallas_call(
        body,
        out_shape=jax.ShapeDtypeStruct((2 * M, NH), jnp.bfloat16),
        in_specs=[pl.BlockSpec(memory_space=pl.ANY)],
        out_specs=pl.BlockSpec(memory_space=pl.ANY),
        scratch_shapes=[
            pltpu.VMEM((H, NH), jnp.float32),
            pltpu.VMEM((H, NH), jnp.bfloat16),
            pltpu.VMEM((M, NH), jnp.float32),
            pltpu.VMEM((M, NH), jnp.bfloat16),
            pltpu.VMEM((H, NH), jnp.bfloat16),
            pltpu.SemaphoreType.DMA((C,)),
            pltpu.SemaphoreType.DMA,
            pltpu.SemaphoreType.DMA,
            pltpu.SemaphoreType.DMA((C,)),
            pltpu.SemaphoreType.DMA((C,)),
            pltpu.SemaphoreType.DMA((C,)),
            pltpu.SemaphoreType.DMA((C,)),
            pltpu.SemaphoreType.DMA((C,)),
        ],
        compiler_params=pltpu.CompilerParams(collective_id=0),
    )(x)


# device time: 16082 ns/iter; 1.0586x vs baseline; 1.0586x over previous
import jax
import jax.numpy as jnp
from jax import lax
from jax.experimental import pallas as pl
from jax.experimental.pallas import tpu as pltpu

M = 1024
N = 1024
NH = 512
H = 512
C = 8
CH = H // C


def kernel(x):
    def body(x_ref, out_ref, send_buf, s1_send, s1_recv, s2_send, s2_recv):
        p = lax.axis_index("x")
        y = lax.axis_index("y")
        z = lax.axis_index("z")
        q = 1 - p
        r = y % 2
        by = y + 1 - 2 * r

        barrier_sem = pltpu.get_barrier_semaphore()
        pl.semaphore_signal(
            barrier_sem, inc=1, device_id=(q, y, z),
            device_id_type=pl.DeviceIdType.MESH,
        )
        pl.semaphore_signal(
            barrier_sem, inc=1, device_id=(p, by, z),
            device_id_type=pl.DeviceIdType.MESH,
        )
        pl.semaphore_wait(barrier_sem, 2)

        rdma1 = []
        for c in range(C):
            row0 = r * H + c * CH

            @pl.when(p == 0)
            def _(row0=row0, c=c):
                send_buf[pl.ds(c * CH, CH)] = x_ref[
                    pl.ds(row0, CH), NH:N
                ].astype(jnp.bfloat16)

            @pl.when(p == 1)
            def _(row0=row0, c=c):
                send_buf[pl.ds(c * CH, CH)] = x_ref[
                    pl.ds(row0, CH), 0:NH
                ].astype(jnp.bfloat16)

            rdma = pltpu.make_async_remote_copy(
                src_ref=send_buf.at[pl.ds(c * CH, CH)],
                dst_ref=out_ref.at[pl.ds(p * M + row0, CH)],
                send_sem=s1_send.at[c],
                recv_sem=s1_recv.at[c],
                device_id=(q, y, z),
                device_id_type=pl.DeviceIdType.MESH,
            )
            rdma.start()
            rdma1.append(rdma)

        @pl.when(p == 0)
        def _():
            out_ref[0:M] = x_ref[:, 0:NH].astype(jnp.bfloat16)

        @pl.when(p == 1)
        def _():
            out_ref[M : 2 * M] = x_ref[:, NH:N].astype(jnp.bfloat16)

        rdma2 = []
        for c in range(C):
            row0 = q * M + r * H + c * CH
            rdma1[c].wait_recv()
            rdma = pltpu.make_async_remote_copy(
                src_ref=out_ref.at[pl.ds(row0, CH)],
                dst_ref=out_ref.at[pl.ds(row0, CH)],
                send_sem=s2_send.at[c],
                recv_sem=s2_recv.at[c],
                device_id=(p, by, z),
                device_id_type=pl.DeviceIdType.MESH,
            )
            rdma.start()
            rdma2.append(rdma)

        for c in range(C):
            rdma1[c].wait_send()
            rdma2[c].wait()

    return pl.pallas_call(
        body,
        out_shape=jax.ShapeDtypeStruct((2 * M, NH), jnp.bfloat16),
        in_specs=[pl.BlockSpec(memory_space=pltpu.VMEM)],
        out_specs=pl.BlockSpec(memory_space=pltpu.VMEM),
        scratch_shapes=[
            pltpu.VMEM((H, NH), jnp.bfloat16),
            pltpu.SemaphoreType.DMA((C,)),
            pltpu.SemaphoreType.DMA((C,)),
            pltpu.SemaphoreType.DMA((C,)),
            pltpu.SemaphoreType.DMA((C,)),
        ],
        compiler_params=pltpu.CompilerParams(collective_id=0),
    )(x)


# device time: 16038 ns/iter; 1.0615x vs baseline; 1.0027x over previous
import jax
import jax.numpy as jnp
from jax import lax
from jax.experimental import pallas as pl
from jax.experimental.pallas import tpu as pltpu

M = 1024
N = 1024
NH = 512
H = 512
C = 16
CH = H // C


def kernel(x):
    def body(x_ref, out_ref, send_buf, s1_send, s1_recv, s2_send, s2_recv):
        p = lax.axis_index("x")
        y = lax.axis_index("y")
        z = lax.axis_index("z")
        q = 1 - p
        r = y % 2
        by = y + 1 - 2 * r

        barrier_sem = pltpu.get_barrier_semaphore()
        pl.semaphore_signal(
            barrier_sem, inc=1, device_id=(q, y, z),
            device_id_type=pl.DeviceIdType.MESH,
        )
        pl.semaphore_signal(
            barrier_sem, inc=1, device_id=(p, by, z),
            device_id_type=pl.DeviceIdType.MESH,
        )
        pl.semaphore_wait(barrier_sem, 2)

        rdma1 = []
        for c in range(C):
            row0 = r * H + c * CH

            @pl.when(p == 0)
            def _(row0=row0, c=c):
                send_buf[pl.ds(c * CH, CH)] = x_ref[
                    pl.ds(row0, CH), NH:N
                ].astype(jnp.bfloat16)

            @pl.when(p == 1)
            def _(row0=row0, c=c):
                send_buf[pl.ds(c * CH, CH)] = x_ref[
                    pl.ds(row0, CH), 0:NH
                ].astype(jnp.bfloat16)

            rdma = pltpu.make_async_remote_copy(
                src_ref=send_buf.at[pl.ds(c * CH, CH)],
                dst_ref=out_ref.at[pl.ds(p * M + row0, CH)],
                send_sem=s1_send.at[c],
                recv_sem=s1_recv.at[c],
                device_id=(q, y, z),
                device_id_type=pl.DeviceIdType.MESH,
            )
            rdma.start()
            rdma1.append(rdma)

        @pl.when(p == 0)
        def _():
            out_ref[0:M] = x_ref[:, 0:NH].astype(jnp.bfloat16)

        @pl.when(p == 1)
        def _():
            out_ref[M : 2 * M] = x_ref[:, NH:N].astype(jnp.bfloat16)

        rdma2 = []
        for c in range(C):
            row0 = q * M + r * H + c * CH
            rdma1[c].wait_recv()
            rdma = pltpu.make_async_remote_copy(
                src_ref=out_ref.at[pl.ds(row0, CH)],
                dst_ref=out_ref.at[pl.ds(row0, CH)],
                send_sem=s2_send.at[c],
                recv_sem=s2_recv.at[c],
                device_id=(p, by, z),
                device_id_type=pl.DeviceIdType.MESH,
            )
            rdma.start()
            rdma2.append(rdma)

        for c in range(C):
            rdma1[c].wait_send()
            rdma2[c].wait()

    return pl.pallas_call(
        body,
        out_shape=jax.ShapeDtypeStruct((2 * M, NH), jnp.bfloat16),
        in_specs=[pl.BlockSpec(memory_space=pltpu.VMEM)],
        out_specs=pl.BlockSpec(memory_space=pltpu.VMEM),
        scratch_shapes=[
            pltpu.VMEM((H, NH), jnp.bfloat16),
            pltpu.SemaphoreType.DMA((C,)),
            pltpu.SemaphoreType.DMA((C,)),
            pltpu.SemaphoreType.DMA((C,)),
            pltpu.SemaphoreType.DMA((C,)),
        ],
        compiler_params=pltpu.CompilerParams(collective_id=0),
    )(x)


# device time: 15634 ns/iter; 1.0889x vs baseline; 1.0258x over previous
import jax
import jax.numpy as jnp
from jax import lax
from jax.experimental import pallas as pl
from jax.experimental.pallas import tpu as pltpu

M = 1024
N = 1024
NH = 512
H = 512
C = 16
CH = H // C
K = 3


def kernel(x):
    def body(
        x_ref,
        out_ref,
        send_buf,
        tail_buf,
        s1_send,
        s1_recv,
        s2_send,
        s2_recv,
        s3_send,
        s3_recv,
    ):
        p = lax.axis_index("x")
        y = lax.axis_index("y")
        z = lax.axis_index("z")
        q = 1 - p
        r = y % 2
        by = y + 1 - 2 * r

        barrier_sem = pltpu.get_barrier_semaphore()
        pl.semaphore_signal(
            barrier_sem, inc=1, device_id=(q, y, z),
            device_id_type=pl.DeviceIdType.MESH,
        )
        pl.semaphore_signal(
            barrier_sem, inc=1, device_id=(p, by, z),
            device_id_type=pl.DeviceIdType.MESH,
        )

        @pl.when(p == 0)
        def _():
            send_buf[...] = x_ref[pl.ds(r * H, H), NH:N].astype(jnp.bfloat16)
            tail_buf[...] = x_ref[
                pl.ds((1 - r) * H + (C - K) * CH, K * CH), NH:N
            ].astype(jnp.bfloat16)

        @pl.when(p == 1)
        def _():
            send_buf[...] = x_ref[pl.ds(r * H, H), 0:NH].astype(jnp.bfloat16)
            tail_buf[...] = x_ref[
                pl.ds((1 - r) * H + (C - K) * CH, K * CH), 0:NH
            ].astype(jnp.bfloat16)

        pl.semaphore_wait(barrier_sem, 2)

        rdma1 = []
        for c in range(C):
            rdma = pltpu.make_async_remote_copy(
                src_ref=send_buf.at[pl.ds(c * CH, CH)],
                dst_ref=out_ref.at[pl.ds(p * M + r * H + c * CH, CH)],
                send_sem=s1_send.at[c],
                recv_sem=s1_recv.at[c],
                device_id=(q, y, z),
                device_id_type=pl.DeviceIdType.MESH,
            )
            rdma.start()
            rdma1.append(rdma)
        rdma3 = pltpu.make_async_remote_copy(
            src_ref=tail_buf,
            dst_ref=out_ref.at[pl.ds(p * M + (1 - r) * H + (C - K) * CH, K * CH)],
            send_sem=s3_send,
            recv_sem=s3_recv,
            device_id=(q, y, z),
            device_id_type=pl.DeviceIdType.MESH,
        )
        rdma3.start()

        @pl.when(p == 0)
        def _():
            out_ref[0:M] = x_ref[:, 0:NH].astype(jnp.bfloat16)

        @pl.when(p == 1)
        def _():
            out_ref[M : 2 * M] = x_ref[:, NH:N].astype(jnp.bfloat16)

        rdma2 = []
        for c in range(C - K):
            row0 = q * M + r * H + c * CH
            rdma1[c].wait_recv()
            rdma = pltpu.make_async_remote_copy(
                src_ref=out_ref.at[pl.ds(row0, CH)],
                dst_ref=out_ref.at[pl.ds(row0, CH)],
                send_sem=s2_send.at[c],
                recv_sem=s2_recv.at[c],
                device_id=(p, by, z),
                device_id_type=pl.DeviceIdType.MESH,
            )
            rdma.start()
            rdma2.append(rdma)

        for c in range(C - K, C):
            rdma1[c].wait_recv()
        for c in range(C):
            rdma1[c].wait_send()
        for rd in rdma2:
            rd.wait()
        rdma3.wait()

    return pl.pallas_call(
        body,
        out_shape=jax.ShapeDtypeStruct((2 * M, NH), jnp.bfloat16),
        in_specs=[pl.BlockSpec(memory_space=pltpu.VMEM)],
        out_specs=pl.BlockSpec(memory_space=pltpu.VMEM),
        scratch_shapes=[
            pltpu.VMEM((H, NH), jnp.bfloat16),
            pltpu.VMEM((K * CH, NH), jnp.bfloat16),
            pltpu.SemaphoreType.DMA((C,)),
            pltpu.SemaphoreType.DMA((C,)),
            pltpu.SemaphoreType.DMA((C,)),
            pltpu.SemaphoreType.DMA((C,)),
            pltpu.SemaphoreType.DMA,
            pltpu.SemaphoreType.DMA,
        ],
        compiler_params=pltpu.CompilerParams(collective_id=0),
    )(x)


# device time: 14394 ns/iter; 1.1827x vs baseline; 1.0861x over previous
import jax
import jax.numpy as jnp
from jax import lax
from jax.experimental import pallas as pl
from jax.experimental.pallas import tpu as pltpu

M = 1024
N = 1024
NH = 512
Q = 256
CQ = 4
QC = Q // CQ


def kernel(x):
    def body(
        x_ref,
        out_ref,
        send_buf,
        s1_send,
        s1_recv,
        s1b_send,
        s1b_recv,
        s2y_send,
        s2y_recv,
        s2z_send,
        s2z_recv,
    ):
        p = lax.axis_index("x")
        y = lax.axis_index("y")
        z = lax.axis_index("z")
        q = 1 - p
        ry = y % 2
        rz = z % 2
        g = 2 * ry + rz
        by = y + 1 - 2 * ry
        bz = z + 1 - 2 * rz

        barrier_sem = pltpu.get_barrier_semaphore()
        for dev in [(q, y, z), (p, by, z), (p, y, bz)]:
            pl.semaphore_signal(
                barrier_sem, inc=1, device_id=dev,
                device_id_type=pl.DeviceIdType.MESH,
            )

        @pl.when(p == 0)
        def _():
            send_buf[0:Q] = x_ref[pl.ds(g * Q, Q), NH:N].astype(jnp.bfloat16)
            send_buf[Q : 2 * Q] = x_ref[pl.ds((3 - g) * Q, Q), NH:N].astype(
                jnp.bfloat16
            )

        @pl.when(p == 1)
        def _():
            send_buf[0:Q] = x_ref[pl.ds(g * Q, Q), 0:NH].astype(jnp.bfloat16)
            send_buf[Q : 2 * Q] = x_ref[pl.ds((3 - g) * Q, Q), 0:NH].astype(
                jnp.bfloat16
            )

        pl.semaphore_wait(barrier_sem, 3)

        rdma1 = []
        for c in range(CQ):
            rdma = pltpu.make_async_remote_copy(
                src_ref=send_buf.at[pl.ds(c * QC, QC)],
                dst_ref=out_ref.at[pl.ds(p * M + g * Q + c * QC, QC)],
                send_sem=s1_send.at[c],
                recv_sem=s1_recv.at[c],
                device_id=(q, y, z),
                device_id_type=pl.DeviceIdType.MESH,
            )
            rdma.start()
            rdma1.append(rdma)
        rdma1b = []
        for c in range(CQ):
            rdma = pltpu.make_async_remote_copy(
                src_ref=send_buf.at[pl.ds(Q + c * QC, QC)],
                dst_ref=out_ref.at[pl.ds(p * M + (3 - g) * Q + c * QC, QC)],
                send_sem=s1b_send.at[c],
                recv_sem=s1b_recv.at[c],
                device_id=(q, y, z),
                device_id_type=pl.DeviceIdType.MESH,
            )
            rdma.start()
            rdma1b.append(rdma)

        @pl.when(p == 0)
        def _():
            out_ref[0:M] = x_ref[:, 0:NH].astype(jnp.bfloat16)

        @pl.when(p == 1)
        def _():
            out_ref[M : 2 * M] = x_ref[:, NH:N].astype(jnp.bfloat16)

        rdma2 = []
        for c in range(CQ):
            row0 = q * M + g * Q + c * QC
            rdma1[c].wait_recv()
            for dev, ssem, rsem in [
                ((p, by, z), s2y_send, s2y_recv),
                ((p, y, bz), s2z_send, s2z_recv),
            ]:
                rdma = pltpu.make_async_remote_copy(
                    src_ref=out_ref.at[pl.ds(row0, QC)],
                    dst_ref=out_ref.at[pl.ds(row0, QC)],
                    send_sem=ssem.at[c],
                    recv_sem=rsem.at[c],
                    device_id=dev,
                    device_id_type=pl.DeviceIdType.MESH,
                )
                rdma.start()
                rdma2.append(rdma)

        for c in range(CQ):
            rdma1b[c].wait_recv()
        for c in range(CQ):
            rdma1[c].wait_send()
            rdma1b[c].wait_send()
        for rd in rdma2:
            rd.wait()

    return pl.pallas_call(
        body,
        out_shape=jax.ShapeDtypeStruct((2 * M, NH), jnp.bfloat16),
        in_specs=[pl.BlockSpec(memory_space=pltpu.VMEM)],
        out_specs=pl.BlockSpec(memory_space=pltpu.VMEM),
        scratch_shapes=[
            pltpu.VMEM((2 * Q, NH), jnp.bfloat16),
            pltpu.SemaphoreType.DMA((CQ,)),
            pltpu.SemaphoreType.DMA((CQ,)),
            pltpu.SemaphoreType.DMA((CQ,)),
            pltpu.SemaphoreType.DMA((CQ,)),
            pltpu.SemaphoreType.DMA((CQ,)),
            pltpu.SemaphoreType.DMA((CQ,)),
            pltpu.SemaphoreType.DMA((CQ,)),
            pltpu.SemaphoreType.DMA((CQ,)),
        ],
        compiler_params=pltpu.CompilerParams(collective_id=0),
    )(x)


# device time: 14376 ns/iter; 1.1842x vs baseline; 1.0013x over previous
import jax
import jax.numpy as jnp
from jax import lax
from jax.experimental import pallas as pl
from jax.experimental.pallas import tpu as pltpu

M = 1024
N = 1024
NH = 512
Q = 256
CQ = 4
QC = Q // CQ


def kernel(x):
    def body(
        x_ref,
        out_ref,
        send_buf,
        s1_send,
        s1_recv,
        s1b_send,
        s1b_recv,
        s2y_send,
        s2y_recv,
        s2z_send,
        s2z_recv,
    ):
        p = lax.axis_index("x")
        y = lax.axis_index("y")
        z = lax.axis_index("z")
        q = 1 - p
        ry = y % 2
        rz = z % 2
        g = 2 * ry + rz
        by = y + 1 - 2 * ry
        bz = z + 1 - 2 * rz

        barrier_sem = pltpu.get_barrier_semaphore()
        for dev in [(q, y, z), (p, by, z), (p, y, bz)]:
            pl.semaphore_signal(
                barrier_sem, inc=1, device_id=dev,
                device_id_type=pl.DeviceIdType.MESH,
            )

        pl.semaphore_wait(barrier_sem, 3)

        rdma1 = []
        for c in range(CQ):

            @pl.when(p == 0)
            def _(c=c):
                send_buf[pl.ds(c * QC, QC)] = x_ref[
                    pl.ds(g * Q + c * QC, QC), NH:N
                ].astype(jnp.bfloat16)

            @pl.when(p == 1)
            def _(c=c):
                send_buf[pl.ds(c * QC, QC)] = x_ref[
                    pl.ds(g * Q + c * QC, QC), 0:NH
                ].astype(jnp.bfloat16)

            rdma = pltpu.make_async_remote_copy(
                src_ref=send_buf.at[pl.ds(c * QC, QC)],
                dst_ref=out_ref.at[pl.ds(p * M + g * Q + c * QC, QC)],
                send_sem=s1_send.at[c],
                recv_sem=s1_recv.at[c],
                device_id=(q, y, z),
                device_id_type=pl.DeviceIdType.MESH,
            )
            rdma.start()
            rdma1.append(rdma)
        rdma1b = []
        for c in range(CQ):

            @pl.when(p == 0)
            def _(c=c):
                send_buf[pl.ds(Q + c * QC, QC)] = x_ref[
                    pl.ds((3 - g) * Q + c * QC, QC), NH:N
                ].astype(jnp.bfloat16)

            @pl.when(p == 1)
            def _(c=c):
                send_buf[pl.ds(Q + c * QC, QC)] = x_ref[
                    pl.ds((3 - g) * Q + c * QC, QC), 0:NH
                ].astype(jnp.bfloat16)

            rdma = pltpu.make_async_remote_copy(
                src_ref=send_buf.at[pl.ds(Q + c * QC, QC)],
                dst_ref=out_ref.at[pl.ds(p * M + (3 - g) * Q + c * QC, QC)],
                send_sem=s1b_send.at[c],
                recv_sem=s1b_recv.at[c],
                device_id=(q, y, z),
                device_id_type=pl.DeviceIdType.MESH,
            )
            rdma.start()
            rdma1b.append(rdma)

        @pl.when(p == 0)
        def _():
            out_ref[0:M] = x_ref[:, 0:NH].astype(jnp.bfloat16)

        @pl.when(p == 1)
        def _():
            out_ref[M : 2 * M] = x_ref[:, NH:N].astype(jnp.bfloat16)

        rdma2 = []
        for c in range(CQ):
            row0 = q * M + g * Q + c * QC
            rdma1[c].wait_recv()
            for dev, ssem, rsem in [
                ((p, by, z), s2y_send, s2y_recv),
                ((p, y, bz), s2z_send, s2z_recv),
            ]:
                rdma = pltpu.make_async_remote_copy(
                    src_ref=out_ref.at[pl.ds(row0, QC)],
                    dst_ref=out_ref.at[pl.ds(row0, QC)],
                    send_sem=ssem.at[c],
                    recv_sem=rsem.at[c],
                    device_id=dev,
                    device_id_type=pl.DeviceIdType.MESH,
                )
                rdma.start()
                rdma2.append(rdma)

        for c in range(CQ):
            rdma1b[c].wait_recv()
        for c in range(CQ):
            rdma1[c].wait_send()
            rdma1b[c].wait_send()
        for rd in rdma2:
            rd.wait()

    return pl.pallas_call(
        body,
        out_shape=jax.ShapeDtypeStruct((2 * M, NH), jnp.bfloat16),
        in_specs=[pl.BlockSpec(memory_space=pltpu.VMEM)],
        out_specs=pl.BlockSpec(memory_space=pltpu.VMEM),
        scratch_shapes=[
            pltpu.VMEM((2 * Q, NH), jnp.bfloat16),
            pltpu.SemaphoreType.DMA((CQ,)),
            pltpu.SemaphoreType.DMA((CQ,)),
            pltpu.SemaphoreType.DMA((CQ,)),
            pltpu.SemaphoreType.DMA((CQ,)),
            pltpu.SemaphoreType.DMA((CQ,)),
            pltpu.SemaphoreType.DMA((CQ,)),
            pltpu.SemaphoreType.DMA((CQ,)),
            pltpu.SemaphoreType.DMA((CQ,)),
        ],
        compiler_params=pltpu.CompilerParams(collective_id=0),
    )(x)
